# baseline (speedup 1.0000x reference)
import functools

import jax
import jax.numpy as jnp
from jax import lax
from jax.experimental import pallas as pl
from jax.experimental.pallas import tpu as pltpu
from jax.experimental.pallas import tpu_sc as plsc

B = 1024
L = 200
D = 128
V = 1000

B_BLK = 128


NC = 2
NS = 16


@functools.cache
def _make_sc_gather():
    nw = NC * NS
    b_per_w = B // nw
    mesh = plsc.VectorSubcoreMesh(
        core_axis_name="c",
        subcore_axis_name="s",
        num_cores=NC,
        num_subcores=NS,
    )

    @functools.partial(
        pl.kernel,
        mesh=mesh,
        out_type=jax.ShapeDtypeStruct((B, D), jnp.float32),
        scratch_types=[
            pltpu.VMEM((b_per_w,), jnp.int32),
            pltpu.VMEM((b_per_w, D), jnp.float32),
            pltpu.SemaphoreType.DMA,
        ],
    )
    def sc_gather(table_hbm, idx_hbm, out_hbm, idx_v, rows_v, sem):
        wid = lax.axis_index("s") * NC + lax.axis_index("c")
        base = wid * b_per_w
        pltpu.sync_copy(idx_hbm.at[pl.ds(base, b_per_w)], idx_v)
        pltpu.async_copy(table_hbm.at[idx_v], rows_v, sem).wait()
        pltpu.sync_copy(rows_v, out_hbm.at[pl.ds(base, b_per_w)])

    return sc_gather


L_BLK = 40


def _concat_body(x_ref, rows_ref, o_ref):
    o_ref[:, :, :D] = x_ref[...]
    rows = rows_ref[...]
    o_ref[:, :, D:] = jnp.broadcast_to(rows[:, None, :], (B_BLK, L_BLK, D))


_concat_call = pl.pallas_call(
    _concat_body,
    grid=(B // B_BLK, L // L_BLK),
    in_specs=[
        pl.BlockSpec((B_BLK, L_BLK, D), lambda i, j: (i, j, 0)),
        pl.BlockSpec((B_BLK, D), lambda i, j: (i, 0)),
    ],
    out_specs=pl.BlockSpec((B_BLK, L_BLK, 2 * D), lambda i, j: (i, j, 0)),
    out_shape=jax.ShapeDtypeStruct((B, L, 2 * D), jnp.float32),
)


@jax.jit
def kernel(x, labels_pointer, emb_table):
    rows = _make_sc_gather()(emb_table, labels_pointer)
    return _concat_call(x, rows)


# speedup vs baseline: 1.0314x; 1.0314x over previous
import functools

import jax
import jax.numpy as jnp
from jax import lax
from jax.experimental import pallas as pl
from jax.experimental.pallas import tpu as pltpu
from jax.experimental.pallas import tpu_sc as plsc

B = 1024
L = 200
D = 128
V = 1000

B_BLK = 64


NC = 2
NS = 16


@functools.cache
def _make_sc_gather():
    nw = NC * NS
    b_per_w = B // nw
    mesh = plsc.VectorSubcoreMesh(
        core_axis_name="c",
        subcore_axis_name="s",
        num_cores=NC,
        num_subcores=NS,
    )

    @functools.partial(
        pl.kernel,
        mesh=mesh,
        out_type=jax.ShapeDtypeStruct((B, D), jnp.float32),
        scratch_types=[
            pltpu.VMEM((b_per_w,), jnp.int32),
            pltpu.VMEM((b_per_w, D), jnp.float32),
            pltpu.SemaphoreType.DMA,
        ],
    )
    def sc_gather(table_hbm, idx_hbm, out_hbm, idx_v, rows_v, sem):
        wid = lax.axis_index("s") * NC + lax.axis_index("c")
        base = wid * b_per_w
        pltpu.sync_copy(idx_hbm.at[pl.ds(base, b_per_w)], idx_v)
        pltpu.async_copy(table_hbm.at[idx_v], rows_v, sem).wait()
        pltpu.sync_copy(rows_v, out_hbm.at[pl.ds(base, b_per_w)])

    return sc_gather


def _concat_body(x_ref, rows_ref, o_ref):
    o_ref[:, :, :D] = x_ref[...]
    rows = rows_ref[...]
    o_ref[:, :, D:] = jnp.broadcast_to(rows[:, None, :], (B_BLK, L, D))


_concat_call = pl.pallas_call(
    _concat_body,
    grid=(B // B_BLK,),
    in_specs=[
        pl.BlockSpec((B_BLK, L, D), lambda i: (i, 0, 0)),
        pl.BlockSpec((B_BLK, D), lambda i: (i, 0)),
    ],
    out_specs=pl.BlockSpec((B_BLK, L, 2 * D), lambda i: (i, 0, 0)),
    out_shape=jax.ShapeDtypeStruct((B, L, 2 * D), jnp.float32),
)


@jax.jit
def kernel(x, labels_pointer, emb_table):
    rows = _make_sc_gather()(emb_table, labels_pointer)
    return _concat_call(x, rows)


# speedup vs baseline: 1.0445x; 1.0127x over previous
import functools

import jax
import jax.numpy as jnp
from jax import lax
from jax.experimental import pallas as pl
from jax.experimental.pallas import tpu as pltpu
from jax.experimental.pallas import tpu_sc as plsc

B = 1024
L = 200
D = 128
V = 1000

B_BLK = 64


NC = 2
NS = 16


@functools.cache
def _make_sc_gather():
    nw = NC * NS
    b_per_w = B // nw
    mesh = plsc.VectorSubcoreMesh(
        core_axis_name="c",
        subcore_axis_name="s",
        num_cores=NC,
        num_subcores=NS,
    )

    @functools.partial(
        pl.kernel,
        mesh=mesh,
        out_type=jax.ShapeDtypeStruct((B, D), jnp.float32),
        scratch_types=[
            pltpu.VMEM((b_per_w,), jnp.int32),
            pltpu.VMEM((b_per_w, D), jnp.float32),
            pltpu.SemaphoreType.DMA,
        ],
    )
    def sc_gather(table_hbm, idx_hbm, out_hbm, idx_v, rows_v, sem):
        wid = lax.axis_index("s") * NC + lax.axis_index("c")
        base = wid * b_per_w
        pltpu.sync_copy(idx_hbm.at[pl.ds(base, b_per_w)], idx_v)
        pltpu.async_copy(table_hbm.at[idx_v], rows_v, sem).wait()
        pltpu.sync_copy(rows_v, out_hbm.at[pl.ds(base, b_per_w)])

    return sc_gather


def _concat_body(x_ref, rows_ref, o_ref):
    i = pl.program_id(0)
    o_ref[:, :, :D] = x_ref[...]
    rows = rows_ref[pl.ds(i * B_BLK, B_BLK), :]
    o_ref[:, :, D:] = jnp.broadcast_to(rows[:, None, :], (B_BLK, L, D))


_concat_call = pl.pallas_call(
    _concat_body,
    grid=(B // B_BLK,),
    in_specs=[
        pl.BlockSpec((B_BLK, L, D), lambda i: (i, 0, 0)),
        pl.BlockSpec((B, D), lambda i: (0, 0)),
    ],
    out_specs=pl.BlockSpec((B_BLK, L, 2 * D), lambda i: (i, 0, 0)),
    out_shape=jax.ShapeDtypeStruct((B, L, 2 * D), jnp.float32),
)


@jax.jit
def kernel(x, labels_pointer, emb_table):
    rows = _make_sc_gather()(emb_table, labels_pointer)
    return _concat_call(x, rows)
